# baseline (speedup 1.0000x reference)
import functools

import jax
import jax.numpy as jnp
from jax import lax
from jax.experimental import pallas as pl
from jax.experimental.pallas import tpu as pltpu
from jax.experimental.pallas import tpu_sc as plsc

P = 4096
SEQ = 4096
STATE = 2048
L = 16
CH = 16
U = 8
BLK = 512


def _make_sc_energies(r0, nrows, we):
    def body(hist_hbm, cur_hbm, dist_hbm, e_hbm,
             hist_v, cur_all_v, idx16_v, rows_v, ea_v, eb_v,
             sem_in, sem_a, sem_b):
        info = plsc.get_sparse_core_info()
        nc, ns = info.num_cores, info.num_subcores
        nw = nc * ns
        wid = lax.axis_index("s") * nc + lax.axis_index("c")

        pltpu.sync_copy(hist_hbm, hist_v)
        pltpu.sync_copy(cur_hbm, cur_all_v)

        iota = lax.iota(jnp.int32, L)

        def gather_row(t, e_ref):
            c = t >> 4
            k = t - (c << 4)
            i = r0 + wid + t * nw
            kvec = jnp.full((L,), k, jnp.int32)

            @pl.when(k == 0)
            def _():
                rowidx = plsc.load_gather(
                    cur_all_v, [(r0 + wid) + (c * CH + iota) * nw])
                idx16_v[pl.ds(0, L)] = rowidx
                pltpu.async_copy(
                    dist_hbm.at[idx16_v], rows_v, sem_in).wait()

            nv2 = (((i + 1) >> 7) << 3) + 16

            @plsc.parallel_loop(0, nv2, unroll=U)
            def _(v):
                idx = hist_v[pl.ds(v * L, L)]
                g = plsc.load_gather(rows_v, [kvec, idx])
                e_ref[pl.ds(v * L, L)] = 1.0 / g

            return i - r0

        def pair_body(q, carry):
            ia = gather_row(2 * q, ea_v)
            pltpu.async_copy(ea_v.at[pl.ds(0, we)],
                             e_hbm.at[ia, pl.ds(0, we)], sem_a)

            @pl.when(q > 0)
            def _():
                pltpu.make_async_copy(eb_v.at[pl.ds(0, we)],
                                      e_hbm.at[ia, pl.ds(0, we)],
                                      sem_b).wait()

            ib = gather_row(2 * q + 1, eb_v)
            pltpu.async_copy(eb_v.at[pl.ds(0, we)],
                             e_hbm.at[ib, pl.ds(0, we)], sem_b)
            pltpu.make_async_copy(ea_v.at[pl.ds(0, we)],
                                  e_hbm.at[ia, pl.ds(0, we)],
                                  sem_a).wait()
            return carry

        lax.fori_loop(0, nrows // nw // 2, pair_body, 0)
        pltpu.make_async_copy(eb_v.at[pl.ds(0, we)],
                              e_hbm.at[0, pl.ds(0, we)], sem_b).wait()

    return functools.partial(
        pl.kernel,
        out_type=jax.ShapeDtypeStruct((nrows, we), jnp.float32),
        mesh=plsc.VectorSubcoreMesh(
            core_axis_name="c", subcore_axis_name="s"),
        compiler_params=pltpu.CompilerParams(
            use_tc_tiling_on_sc=True, needs_layout_passes=False),
        scratch_types=[
            pltpu.VMEM((SEQ,), jnp.int32),
            pltpu.VMEM((STATE,), jnp.int32),
            pltpu.VMEM((L,), jnp.int32),
            pltpu.VMEM((CH, SEQ), jnp.float32),
            pltpu.VMEM((SEQ,), jnp.float32),
            pltpu.VMEM((SEQ,), jnp.float32),
            pltpu.SemaphoreType.DMA,
            pltpu.SemaphoreType.DMA,
            pltpu.SemaphoreType.DMA,
        ],
    )(body)


_sc_low = _make_sc_energies(0, 1024, 1024)
_sc_high = _make_sc_energies(1024, 1024, 2048)


def _make_tc_body(w, r0):
    ntail = float(SEQ - w)

    def body(e_ref, *rest):
        o_ref = rest[-1]
        b = pl.program_id(0)
        rows = (jax.lax.broadcasted_iota(jnp.int32, (BLK, w), 0)
                + b * BLK + r0)
        cols = jax.lax.broadcasted_iota(jnp.int32, (BLK, w), 1)
        e = jnp.where(cols <= rows, e_ref[...], 0.0)
        m = jnp.max(e, axis=1, keepdims=True)
        p = jnp.exp(e - m)
        em = jnp.exp(-m)
        s = jnp.sum(p, axis=1, keepdims=True) + ntail * em
        r = 1.0 / s
        o_ref[:, :w] = p * r
        o_ref[:, w:] = jnp.broadcast_to(em * r, (BLK, SEQ - w))

    return body


def _tc_softmax_half(e, w, r0, prev):
    off = r0 // BLK
    in_specs = [pl.BlockSpec((BLK, w), lambda b: (b, 0))]
    operands = [e]
    aliases = {}
    if prev is not None:
        in_specs.append(pl.BlockSpec(memory_space=pl.ANY))
        operands.append(prev)
        aliases = {1: 0}
    return pl.pallas_call(
        _make_tc_body(w, r0),
        grid=(1024 // BLK,),
        in_specs=in_specs,
        out_specs=pl.BlockSpec((BLK, SEQ), lambda b: (b + off, 0)),
        out_shape=jax.ShapeDtypeStruct((STATE, SEQ), jnp.float32),
        input_output_aliases=aliases,
    )(*operands)


def kernel(history, current, poi_distance_matrix):
    hist = history.astype(jnp.int32)
    cur = current.astype(jnp.int32)
    e0 = _sc_low(hist, cur, poi_distance_matrix)
    e1 = _sc_high(hist, cur, poi_distance_matrix)
    out = _tc_softmax_half(e0, 1024, 0, None)
    return _tc_softmax_half(e1, 2048, 1024, out)


# speedup vs baseline: 1.0467x; 1.0467x over previous
import functools

import jax
import jax.numpy as jnp
from jax import lax
from jax.experimental import pallas as pl
from jax.experimental.pallas import tpu as pltpu
from jax.experimental.pallas import tpu_sc as plsc

P = 4096
SEQ = 4096
STATE = 2048
EW = 2048
L = 16
CH = 16
U = 8
BLK = 256


def _sc_body(hist_hbm, cur_hbm, dist_hbm, e_hbm,
             hist_v, cur_all_v, idx16_v, rows_v, ea_v, eb_v,
             sem_in, sem_a, sem_b):
    info = plsc.get_sparse_core_info()
    nc, ns = info.num_cores, info.num_subcores
    nw = nc * ns
    wid = lax.axis_index("s") * nc + lax.axis_index("c")

    pltpu.sync_copy(hist_hbm, hist_v)
    pltpu.sync_copy(cur_hbm, cur_all_v)

    iota = lax.iota(jnp.int32, L)

    def gather_row(t, e_ref):
        c = t >> 4
        k = t - (c << 4)
        i = wid + t * nw
        kvec = jnp.full((L,), k, jnp.int32)

        @pl.when(k == 0)
        def _():
            rowidx = plsc.load_gather(
                cur_all_v, [wid + (c * CH + iota) * nw])
            idx16_v[pl.ds(0, L)] = rowidx
            pltpu.async_copy(dist_hbm.at[idx16_v], rows_v, sem_in).wait()

        nv2 = (((i + 1) >> 7) << 3) + 16

        @plsc.parallel_loop(0, nv2, unroll=U)
        def _(v):
            idx = hist_v[pl.ds(v * L, L)]
            g = plsc.load_gather(rows_v, [kvec, idx])
            e_ref[pl.ds(v * L, L)] = 1.0 / g

        return i

    def put_row(i, e_ref, sem):
        @pl.when(i < 1024)
        def _():
            pltpu.async_copy(e_ref.at[pl.ds(0, 1024)],
                             e_hbm.at[i, pl.ds(0, 1024)], sem)

        @pl.when(i >= 1024)
        def _():
            pltpu.async_copy(e_ref.at[pl.ds(0, 2048)],
                             e_hbm.at[i, pl.ds(0, 2048)], sem)

    def drain_row(i, e_ref, sem):
        @pl.when(i < 1024)
        def _():
            pltpu.make_async_copy(e_ref.at[pl.ds(0, 1024)],
                                  e_hbm.at[i, pl.ds(0, 1024)], sem).wait()

        @pl.when(i >= 1024)
        def _():
            pltpu.make_async_copy(e_ref.at[pl.ds(0, 2048)],
                                  e_hbm.at[i, pl.ds(0, 2048)], sem).wait()

    def pair_body(q, carry):
        ia = gather_row(2 * q, ea_v)
        put_row(ia, ea_v, sem_a)

        @pl.when(q > 0)
        def _():
            drain_row(ia - nw, eb_v, sem_b)

        ib = gather_row(2 * q + 1, eb_v)
        put_row(ib, eb_v, sem_b)
        drain_row(ia, ea_v, sem_a)
        return carry

    lax.fori_loop(0, STATE // nw // 2, pair_body, 0)
    pltpu.make_async_copy(eb_v.at[pl.ds(0, 2048)],
                          e_hbm.at[0, pl.ds(0, 2048)], sem_b).wait()


_sc_energies = functools.partial(
    pl.kernel,
    out_type=jax.ShapeDtypeStruct((STATE, EW), jnp.float32),
    mesh=plsc.VectorSubcoreMesh(core_axis_name="c", subcore_axis_name="s"),
    compiler_params=pltpu.CompilerParams(
        use_tc_tiling_on_sc=True, needs_layout_passes=False),
    scratch_types=[
        pltpu.VMEM((SEQ,), jnp.int32),
        pltpu.VMEM((STATE,), jnp.int32),
        pltpu.VMEM((L,), jnp.int32),
        pltpu.VMEM((CH, SEQ), jnp.float32),
        pltpu.VMEM((SEQ,), jnp.float32),
        pltpu.VMEM((SEQ,), jnp.float32),
        pltpu.SemaphoreType.DMA,
        pltpu.SemaphoreType.DMA,
        pltpu.SemaphoreType.DMA,
    ],
)(_sc_body)


def _make_tc_body(w, r0):
    ntail = float(SEQ - w)

    def body(e_ref, *rest):
        o_ref = rest[-1]
        b = pl.program_id(0)
        rows = (jax.lax.broadcasted_iota(jnp.int32, (BLK, w), 0)
                + b * BLK + r0)
        cols = jax.lax.broadcasted_iota(jnp.int32, (BLK, w), 1)
        e = jnp.where(cols <= rows, e_ref[...], 0.0)
        m = jnp.max(e, axis=1, keepdims=True)
        p = jnp.exp(e - m)
        em = jnp.exp(-m)
        s = jnp.sum(p, axis=1, keepdims=True) + ntail * em
        r = 1.0 / s
        o_ref[:, :w] = p * r
        o_ref[:, w:] = jnp.broadcast_to(em * r, (BLK, SEQ - w))

    return body


def _tc_softmax(e):
    return pl.pallas_call(
        _make_tc_body(EW, 0),
        grid=(STATE // BLK,),
        in_specs=[pl.BlockSpec((BLK, EW), lambda b: (b, 0))],
        out_specs=pl.BlockSpec((BLK, SEQ), lambda b: (b, 0)),
        out_shape=jax.ShapeDtypeStruct((STATE, SEQ), jnp.float32),
    )(e)


def kernel(history, current, poi_distance_matrix):
    hist = history.astype(jnp.int32)
    cur = current.astype(jnp.int32)
    e = _sc_energies(hist, cur, poi_distance_matrix)
    return _tc_softmax(e)


# speedup vs baseline: 1.0627x; 1.0152x over previous
import functools

import jax
import jax.numpy as jnp
from jax import lax
from jax.experimental import pallas as pl
from jax.experimental.pallas import tpu as pltpu
from jax.experimental.pallas import tpu_sc as plsc

P = 4096
SEQ = 4096
STATE = 2048
EW = 2048
L = 16
CH = 16
U = 8
BLK = 512


def _sc_body(hist_hbm, cur_hbm, dist_hbm, e_hbm,
             hist_v, cur_all_v, idx16_v, rows_v, ea_v, eb_v,
             sem_in, sem_a, sem_b):
    info = plsc.get_sparse_core_info()
    nc, ns = info.num_cores, info.num_subcores
    nw = nc * ns
    wid = lax.axis_index("s") * nc + lax.axis_index("c")

    pltpu.sync_copy(hist_hbm, hist_v)
    pltpu.sync_copy(cur_hbm, cur_all_v)

    iota = lax.iota(jnp.int32, L)

    def gather_row(t, e_ref):
        c = t >> 4
        k = t - (c << 4)
        i = wid + t * nw
        kvec = jnp.full((L,), k, jnp.int32)

        @pl.when(k == 0)
        def _():
            rowidx = plsc.load_gather(
                cur_all_v, [wid + (c * CH + iota) * nw])
            idx16_v[pl.ds(0, L)] = rowidx
            pltpu.async_copy(dist_hbm.at[idx16_v], rows_v, sem_in).wait()

        nv2 = (((i + 1) >> 7) << 3) + 16

        @plsc.parallel_loop(0, nv2, unroll=U)
        def _(v):
            idx = hist_v[pl.ds(v * L, L)]
            g = plsc.load_gather(rows_v, [kvec, idx])
            e_ref[pl.ds(v * L, L)] = 1.0 / g

        return i

    def put_row(i, e_ref, sem):
        @pl.when(i < 1024)
        def _():
            pltpu.async_copy(e_ref.at[pl.ds(0, 1024)],
                             e_hbm.at[i, pl.ds(0, 1024)], sem)

        @pl.when(i >= 1024)
        def _():
            pltpu.async_copy(e_ref.at[pl.ds(0, 2048)],
                             e_hbm.at[i, pl.ds(0, 2048)], sem)

    def drain_row(i, e_ref, sem):
        @pl.when(i < 1024)
        def _():
            pltpu.make_async_copy(e_ref.at[pl.ds(0, 1024)],
                                  e_hbm.at[i, pl.ds(0, 1024)], sem).wait()

        @pl.when(i >= 1024)
        def _():
            pltpu.make_async_copy(e_ref.at[pl.ds(0, 2048)],
                                  e_hbm.at[i, pl.ds(0, 2048)], sem).wait()

    def pair_body(q, carry):
        ia = gather_row(2 * q, ea_v)
        put_row(ia, ea_v, sem_a)

        @pl.when(q > 0)
        def _():
            drain_row(ia - nw, eb_v, sem_b)

        ib = gather_row(2 * q + 1, eb_v)
        put_row(ib, eb_v, sem_b)
        drain_row(ia, ea_v, sem_a)
        return carry

    lax.fori_loop(0, STATE // nw // 2, pair_body, 0)
    pltpu.make_async_copy(eb_v.at[pl.ds(0, 2048)],
                          e_hbm.at[0, pl.ds(0, 2048)], sem_b).wait()


_sc_energies = functools.partial(
    pl.kernel,
    out_type=jax.ShapeDtypeStruct((STATE, EW), jnp.float32),
    mesh=plsc.VectorSubcoreMesh(core_axis_name="c", subcore_axis_name="s"),
    compiler_params=pltpu.CompilerParams(
        use_tc_tiling_on_sc=True, needs_layout_passes=False),
    scratch_types=[
        pltpu.VMEM((SEQ,), jnp.int32),
        pltpu.VMEM((STATE,), jnp.int32),
        pltpu.VMEM((L,), jnp.int32),
        pltpu.VMEM((CH, SEQ), jnp.float32),
        pltpu.VMEM((SEQ,), jnp.float32),
        pltpu.VMEM((SEQ,), jnp.float32),
        pltpu.SemaphoreType.DMA,
        pltpu.SemaphoreType.DMA,
        pltpu.SemaphoreType.DMA,
    ],
)(_sc_body)


def _make_tc_body(w, r0):
    ntail = float(SEQ - w)

    def body(e_ref, *rest):
        o_ref = rest[-1]
        b = pl.program_id(0)
        rows = (jax.lax.broadcasted_iota(jnp.int32, (BLK, w), 0)
                + b * BLK + r0)
        cols = jax.lax.broadcasted_iota(jnp.int32, (BLK, w), 1)
        e = jnp.where(cols <= rows, e_ref[...], 0.0)
        m = jnp.max(e, axis=1, keepdims=True)
        p = jnp.exp(e - m)
        em = jnp.exp(-m)
        s = jnp.sum(p, axis=1, keepdims=True) + ntail * em
        r = 1.0 / s
        o_ref[:, :w] = p * r
        o_ref[:, w:] = jnp.broadcast_to(em * r, (BLK, SEQ - w))

    return body


def _tc_softmax(e):
    return pl.pallas_call(
        _make_tc_body(EW, 0),
        grid=(STATE // BLK,),
        in_specs=[pl.BlockSpec((BLK, EW), lambda b: (b, 0))],
        out_specs=pl.BlockSpec((BLK, SEQ), lambda b: (b, 0)),
        out_shape=jax.ShapeDtypeStruct((STATE, SEQ), jnp.float32),
    )(e)


def kernel(history, current, poi_distance_matrix):
    hist = history.astype(jnp.int32)
    cur = current.astype(jnp.int32)
    e = _sc_energies(hist, cur, poi_distance_matrix)
    return _tc_softmax(e)


# speedup vs baseline: 1.1207x; 1.0546x over previous
import functools

import jax
import jax.numpy as jnp
from jax import lax
from jax.experimental import pallas as pl
from jax.experimental.pallas import tpu as pltpu
from jax.experimental.pallas import tpu_sc as plsc

P = 4096
SEQ = 4096
STATE = 2048
EW = 2048
L = 16
CH = 8
U = 8
BLK = 512


def _sc_body(hist_hbm, cur_hbm, dist_hbm, e_hbm,
             hist_v, cur_all_v, idxa_v, idxb_v, rows_v, ea_v, eb_v,
             sem_in, sem_a, sem_b):
    info = plsc.get_sparse_core_info()
    nc, ns = info.num_cores, info.num_subcores
    nw = nc * ns
    wid = lax.axis_index("s") * nc + lax.axis_index("c")

    pltpu.sync_copy(hist_hbm, hist_v)
    pltpu.sync_copy(cur_hbm, cur_all_v)

    iota = lax.iota(jnp.int32, L)
    NCH = STATE // 32 // CH

    def stage_chunk(c, idx_ref, buf):
        tt = jnp.minimum(c * CH + iota, STATE // nw - 1)
        rowidx = plsc.load_gather(cur_all_v, [wid + tt * nw])
        idx_ref[pl.ds(0, L)] = rowidx
        pltpu.async_copy(dist_hbm.at[idx_ref.at[pl.ds(0, CH)]], buf, sem_in)

    stage_chunk(0, idxa_v, rows_v.at[0])

    def gather_row(t, e_ref):
        c = lax.div(t, CH)
        k = t - c * CH
        i = wid + t * nw
        kvec = jnp.full((L,), k, jnp.int32)
        pvec = jnp.full((L,), c & 1, jnp.int32)

        @pl.when(k == 0)
        def _():
            pltpu.make_async_copy(
                dist_hbm.at[idxa_v.at[pl.ds(0, CH)]], rows_v.at[0],
                sem_in).wait()

            @pl.when((c < NCH - 1) & ((c & 1) == 0))
            def _():
                stage_chunk(c + 1, idxb_v, rows_v.at[1])

            @pl.when((c < NCH - 1) & ((c & 1) == 1))
            def _():
                stage_chunk(c + 1, idxa_v, rows_v.at[0])

        nv2 = (((i + 1) >> 7) << 3) + 16

        @plsc.parallel_loop(0, nv2, unroll=U)
        def _(v):
            idx = hist_v[pl.ds(v * L, L)]
            g = plsc.load_gather(rows_v, [pvec, kvec, idx])
            e_ref[pl.ds(v * L, L)] = 1.0 / g

        return i

    def put_row(i, e_ref, sem):
        @pl.when(i < 1024)
        def _():
            pltpu.async_copy(e_ref.at[pl.ds(0, 1024)],
                             e_hbm.at[i, pl.ds(0, 1024)], sem)

        @pl.when(i >= 1024)
        def _():
            pltpu.async_copy(e_ref.at[pl.ds(0, 2048)],
                             e_hbm.at[i, pl.ds(0, 2048)], sem)

    def drain_row(i, e_ref, sem):
        @pl.when(i < 1024)
        def _():
            pltpu.make_async_copy(e_ref.at[pl.ds(0, 1024)],
                                  e_hbm.at[i, pl.ds(0, 1024)], sem).wait()

        @pl.when(i >= 1024)
        def _():
            pltpu.make_async_copy(e_ref.at[pl.ds(0, 2048)],
                                  e_hbm.at[i, pl.ds(0, 2048)], sem).wait()

    def pair_body(q, carry):
        ia = gather_row(2 * q, ea_v)
        put_row(ia, ea_v, sem_a)

        @pl.when(q > 0)
        def _():
            drain_row(ia - nw, eb_v, sem_b)

        ib = gather_row(2 * q + 1, eb_v)
        put_row(ib, eb_v, sem_b)
        drain_row(ia, ea_v, sem_a)
        return carry

    lax.fori_loop(0, STATE // nw // 2, pair_body, 0)
    pltpu.make_async_copy(eb_v.at[pl.ds(0, 2048)],
                          e_hbm.at[0, pl.ds(0, 2048)], sem_b).wait()


_sc_energies = functools.partial(
    pl.kernel,
    out_type=jax.ShapeDtypeStruct((STATE, EW), jnp.float32),
    mesh=plsc.VectorSubcoreMesh(core_axis_name="c", subcore_axis_name="s"),
    compiler_params=pltpu.CompilerParams(
        use_tc_tiling_on_sc=True, needs_layout_passes=False),
    scratch_types=[
        pltpu.VMEM((SEQ,), jnp.int32),
        pltpu.VMEM((STATE,), jnp.int32),
        pltpu.VMEM((L,), jnp.int32),
        pltpu.VMEM((L,), jnp.int32),
        pltpu.VMEM((2, CH, SEQ), jnp.float32),
        pltpu.VMEM((SEQ,), jnp.float32),
        pltpu.VMEM((SEQ,), jnp.float32),
        pltpu.SemaphoreType.DMA,
        pltpu.SemaphoreType.DMA,
        pltpu.SemaphoreType.DMA,
    ],
)(_sc_body)


def _make_tc_body(w, r0):
    ntail = float(SEQ - w)

    def body(e_ref, *rest):
        o_ref = rest[-1]
        b = pl.program_id(0)
        rows = (jax.lax.broadcasted_iota(jnp.int32, (BLK, w), 0)
                + b * BLK + r0)
        cols = jax.lax.broadcasted_iota(jnp.int32, (BLK, w), 1)
        e = jnp.where(cols <= rows, e_ref[...], 0.0)
        m = jnp.max(e, axis=1, keepdims=True)
        p = jnp.exp(e - m)
        em = jnp.exp(-m)
        s = jnp.sum(p, axis=1, keepdims=True) + ntail * em
        r = 1.0 / s
        o_ref[:, :w] = p * r
        o_ref[:, w:] = jnp.broadcast_to(em * r, (BLK, SEQ - w))

    return body


def _tc_softmax(e):
    return pl.pallas_call(
        _make_tc_body(EW, 0),
        grid=(STATE // BLK,),
        in_specs=[pl.BlockSpec((BLK, EW), lambda b: (b, 0))],
        out_specs=pl.BlockSpec((BLK, SEQ), lambda b: (b, 0)),
        out_shape=jax.ShapeDtypeStruct((STATE, SEQ), jnp.float32),
    )(e)


def kernel(history, current, poi_distance_matrix):
    hist = history.astype(jnp.int32)
    cur = current.astype(jnp.int32)
    e = _sc_energies(hist, cur, poi_distance_matrix)
    return _tc_softmax(e)


# speedup vs baseline: 1.1401x; 1.0173x over previous
import functools

import jax
import jax.numpy as jnp
from jax import lax
from jax.experimental import pallas as pl
from jax.experimental.pallas import tpu as pltpu
from jax.experimental.pallas import tpu_sc as plsc

P = 4096
SEQ = 4096
STATE = 2048
EW = 2048
L = 16
CH = 8
U = 8
BLK = 512


def _sc_body(hist_hbm, cur_hbm, dist_hbm, e_hbm,
             hist_v, cur_all_v, idxa_v, idxb_v, rows_v, ea_v, eb_v,
             sem_in, sem_a, sem_b):
    info = plsc.get_sparse_core_info()
    nc, ns = info.num_cores, info.num_subcores
    nw = nc * ns
    wid = lax.axis_index("s") * nc + lax.axis_index("c")

    pltpu.sync_copy(hist_hbm, hist_v)
    pltpu.sync_copy(cur_hbm, cur_all_v)

    iota = lax.iota(jnp.int32, L)
    NCH = STATE // 32 // CH

    def stage_chunk(c, idx_ref, buf):
        tt = jnp.minimum(c * CH + iota, STATE // nw - 1)
        rowidx = plsc.load_gather(cur_all_v, [wid + tt * nw])
        idx_ref[pl.ds(0, L)] = rowidx
        pltpu.async_copy(dist_hbm.at[idx_ref.at[pl.ds(0, CH)]], buf, sem_in)

    stage_chunk(0, idxa_v, rows_v.at[0])

    def gather_row(t, e_ref):
        c = lax.div(t, CH)
        k = t - c * CH
        i = wid + t * nw
        kvec = jnp.full((L,), k, jnp.int32)
        pvec = jnp.full((L,), c & 1, jnp.int32)

        @pl.when(k == 0)
        def _():
            pltpu.make_async_copy(
                dist_hbm.at[idxa_v.at[pl.ds(0, CH)]], rows_v.at[0],
                sem_in).wait()

            @pl.when((c < NCH - 1) & ((c & 1) == 0))
            def _():
                stage_chunk(c + 1, idxb_v, rows_v.at[1])

            @pl.when((c < NCH - 1) & ((c & 1) == 1))
            def _():
                stage_chunk(c + 1, idxa_v, rows_v.at[0])

        nv2 = (((i + 1) >> 7) << 3) + 16

        @plsc.parallel_loop(0, nv2, unroll=U)
        def _(v):
            idx = hist_v[pl.ds(v * L, L)]
            g = plsc.load_gather(rows_v, [pvec, kvec, idx])
            e_ref[pl.ds(v * L, L)] = 1.0 / g

        return i

    def put_row(i, e_ref, sem):
        @pl.when(i < 1024)
        def _():
            pltpu.async_copy(e_ref.at[pl.ds(0, 1024)],
                             e_hbm.at[i, pl.ds(0, 1024)], sem)

        @pl.when(i >= 1024)
        def _():
            pltpu.async_copy(e_ref.at[pl.ds(0, 2048)],
                             e_hbm.at[i, pl.ds(0, 2048)], sem)

    def drain_row(i, e_ref, sem):
        @pl.when(i < 1024)
        def _():
            pltpu.make_async_copy(e_ref.at[pl.ds(0, 1024)],
                                  e_hbm.at[i, pl.ds(0, 1024)], sem).wait()

        @pl.when(i >= 1024)
        def _():
            pltpu.make_async_copy(e_ref.at[pl.ds(0, 2048)],
                                  e_hbm.at[i, pl.ds(0, 2048)], sem).wait()

    def pair_body(q, carry):
        ia = gather_row(2 * q, ea_v)
        put_row(ia, ea_v, sem_a)

        @pl.when(q > 0)
        def _():
            drain_row(ia - nw, eb_v, sem_b)

        ib = gather_row(2 * q + 1, eb_v)
        put_row(ib, eb_v, sem_b)
        drain_row(ia, ea_v, sem_a)
        return carry

    lax.fori_loop(0, STATE // nw // 2, pair_body, 0)
    pltpu.make_async_copy(eb_v.at[pl.ds(0, 2048)],
                          e_hbm.at[0, pl.ds(0, 2048)], sem_b).wait()


_sc_energies = functools.partial(
    pl.kernel,
    out_type=jax.ShapeDtypeStruct((STATE, EW), jnp.float32),
    mesh=plsc.VectorSubcoreMesh(core_axis_name="c", subcore_axis_name="s"),
    compiler_params=pltpu.CompilerParams(
        use_tc_tiling_on_sc=True, needs_layout_passes=False),
    scratch_types=[
        pltpu.VMEM((SEQ,), jnp.int32),
        pltpu.VMEM((STATE,), jnp.int32),
        pltpu.VMEM((L,), jnp.int32),
        pltpu.VMEM((L,), jnp.int32),
        pltpu.VMEM((2, CH, SEQ), jnp.float32),
        pltpu.VMEM((SEQ,), jnp.float32),
        pltpu.VMEM((SEQ,), jnp.float32),
        pltpu.SemaphoreType.DMA,
        pltpu.SemaphoreType.DMA,
        pltpu.SemaphoreType.DMA,
    ],
)(_sc_body)


def _tc_softmax_body(e_ref, o_ref):
    b = pl.program_id(0)
    for j in range(STATE // BLK):
        @pl.when(b == j)
        def _(j=j):
            w = (j + 1) * BLK
            rows = jax.lax.broadcasted_iota(jnp.int32, (BLK, BLK), 0) + j * BLK
            cols = jax.lax.broadcasted_iota(jnp.int32, (BLK, BLK), 1) + j * BLK
            strip = jnp.where(cols <= rows, e_ref[:, j * BLK:w], 0.0)
            if j > 0:
                e = jnp.concatenate([e_ref[:, :j * BLK], strip], axis=1)
            else:
                e = strip
            m = jnp.max(e, axis=1, keepdims=True)
            p = jnp.exp(e - m)
            em = jnp.exp(-m)
            s = jnp.sum(p, axis=1, keepdims=True) + float(SEQ - w) * em
            r = 1.0 / s
            o_ref[:, :w] = p * r
            o_ref[:, w:] = jnp.broadcast_to(em * r, (BLK, SEQ - w))


def _tc_softmax(e):
    return pl.pallas_call(
        _tc_softmax_body,
        grid=(STATE // BLK,),
        in_specs=[pl.BlockSpec((BLK, EW), lambda b: (b, 0))],
        out_specs=pl.BlockSpec((BLK, SEQ), lambda b: (b, 0)),
        out_shape=jax.ShapeDtypeStruct((STATE, SEQ), jnp.float32),
    )(e)


def kernel(history, current, poi_distance_matrix):
    hist = history.astype(jnp.int32)
    cur = current.astype(jnp.int32)
    e = _sc_energies(hist, cur, poi_distance_matrix)
    return _tc_softmax(e)


# speedup vs baseline: 1.1548x; 1.0129x over previous
import functools

import jax
import jax.numpy as jnp
from jax import lax
from jax.experimental import pallas as pl
from jax.experimental.pallas import tpu as pltpu
from jax.experimental.pallas import tpu_sc as plsc

P = 4096
SEQ = 4096
STATE = 2048
EW = 2048
L = 16
CH = 8
U = 8
BLK = 512


def _sc_body(hist_hbm, cur_hbm, dist_hbm, e_hbm,
             hist_v, cur_all_v, idxa_v, idxb_v, rows_v, ea_v, eb_v,
             sem_in, sem_a, sem_b):
    info = plsc.get_sparse_core_info()
    nc, ns = info.num_cores, info.num_subcores
    nw = nc * ns
    wid = lax.axis_index("s") * nc + lax.axis_index("c")

    pltpu.sync_copy(hist_hbm, hist_v)
    pltpu.sync_copy(cur_hbm, cur_all_v)

    iota = lax.iota(jnp.int32, L)
    NCH = STATE // 32 // CH

    def stage_chunk(c, idx_ref, buf):
        tt = jnp.minimum(c * CH + iota, STATE // nw - 1)
        rowidx = plsc.load_gather(cur_all_v, [wid + tt * nw])
        idx_ref[pl.ds(0, L)] = rowidx
        pltpu.async_copy(dist_hbm.at[idx_ref.at[pl.ds(0, CH)]], buf, sem_in)

    stage_chunk(0, idxa_v, rows_v.at[0])

    def gather_row(t, e_ref):
        c = lax.div(t, CH)
        k = t - c * CH
        i = wid + t * nw
        kvec = jnp.full((L,), k, jnp.int32)
        pvec = jnp.full((L,), c & 1, jnp.int32)

        @pl.when(k == 0)
        def _():
            pltpu.make_async_copy(
                dist_hbm.at[idxa_v.at[pl.ds(0, CH)]], rows_v.at[0],
                sem_in).wait()

            @pl.when((c < NCH - 1) & ((c & 1) == 0))
            def _():
                stage_chunk(c + 1, idxb_v, rows_v.at[1])

            @pl.when((c < NCH - 1) & ((c & 1) == 1))
            def _():
                stage_chunk(c + 1, idxa_v, rows_v.at[0])

        nv2 = (((i + 1) >> 7) << 3) + 16

        @plsc.parallel_loop(0, nv2, unroll=U)
        def _(v):
            idx = hist_v[pl.ds(v * L, L)]
            g = plsc.load_gather(rows_v, [pvec, kvec, idx])
            e_ref[pl.ds(v * L, L)] = 1.0 / g

        return i

    def put_row(i, e_ref, sem):
        @pl.when(i < 1024)
        def _():
            pltpu.async_copy(e_ref.at[pl.ds(0, 1024)],
                             e_hbm.at[i, pl.ds(0, 1024)], sem)

        @pl.when(i >= 1024)
        def _():
            pltpu.async_copy(e_ref.at[pl.ds(0, 2048)],
                             e_hbm.at[i, pl.ds(0, 2048)], sem)

    def drain_row(i, e_ref, sem):
        @pl.when(i < 1024)
        def _():
            pltpu.make_async_copy(e_ref.at[pl.ds(0, 1024)],
                                  e_hbm.at[i, pl.ds(0, 1024)], sem).wait()

        @pl.when(i >= 1024)
        def _():
            pltpu.make_async_copy(e_ref.at[pl.ds(0, 2048)],
                                  e_hbm.at[i, pl.ds(0, 2048)], sem).wait()

    def pair_body(q, carry):
        ia = gather_row(2 * q, ea_v)
        put_row(ia, ea_v, sem_a)

        @pl.when(q > 0)
        def _():
            drain_row(ia - nw, eb_v, sem_b)

        ib = gather_row(2 * q + 1, eb_v)
        put_row(ib, eb_v, sem_b)
        drain_row(ia, ea_v, sem_a)
        return carry

    lax.fori_loop(0, STATE // nw // 2, pair_body, 0)
    pltpu.make_async_copy(eb_v.at[pl.ds(0, 2048)],
                          e_hbm.at[0, pl.ds(0, 2048)], sem_b).wait()


_sc_energies = functools.partial(
    pl.kernel,
    out_type=jax.ShapeDtypeStruct((STATE, EW), jnp.float32),
    mesh=plsc.VectorSubcoreMesh(core_axis_name="c", subcore_axis_name="s"),
    compiler_params=pltpu.CompilerParams(
        use_tc_tiling_on_sc=True, needs_layout_passes=False),
    scratch_types=[
        pltpu.VMEM((SEQ,), jnp.int32),
        pltpu.VMEM((STATE,), jnp.int32),
        pltpu.VMEM((L,), jnp.int32),
        pltpu.VMEM((L,), jnp.int32),
        pltpu.VMEM((2, CH, SEQ), jnp.float32),
        pltpu.VMEM((SEQ,), jnp.float32),
        pltpu.VMEM((SEQ,), jnp.float32),
        pltpu.SemaphoreType.DMA,
        pltpu.SemaphoreType.DMA,
        pltpu.SemaphoreType.DMA,
    ],
)(_sc_body)


def _tc_softmax_body(e0_ref, e1_ref, e2_ref, e3_ref, o_ref):
    b = pl.program_id(0)
    refs = [e0_ref, e1_ref, e2_ref, e3_ref]
    for j in range(STATE // BLK):
        @pl.when(b == j)
        def _(j=j):
            w = (j + 1) * BLK
            rows = jax.lax.broadcasted_iota(jnp.int32, (BLK, BLK), 0) + j * BLK
            cols = jax.lax.broadcasted_iota(jnp.int32, (BLK, BLK), 1) + j * BLK
            strip = jnp.where(cols <= rows, refs[j][...], 0.0)
            if j > 0:
                e = jnp.concatenate(
                    [refs[t][...] for t in range(j)] + [strip], axis=1)
            else:
                e = strip
            m = jnp.max(e, axis=1, keepdims=True)
            p = jnp.exp(e - m)
            em = jnp.exp(-m)
            s = jnp.sum(p, axis=1, keepdims=True) + float(SEQ - w) * em
            r = 1.0 / s
            o_ref[:, :w] = p * r
            o_ref[:, w:] = jnp.broadcast_to(em * r, (BLK, SEQ - w))


def _make_strip_spec(j):
    return pl.BlockSpec((BLK, BLK), lambda b: (jnp.maximum(b, j), j))


def _tc_softmax(e):
    return pl.pallas_call(
        _tc_softmax_body,
        grid=(STATE // BLK,),
        in_specs=[_make_strip_spec(j) for j in range(STATE // BLK)],
        out_specs=pl.BlockSpec((BLK, SEQ), lambda b: (b, 0)),
        out_shape=jax.ShapeDtypeStruct((STATE, SEQ), jnp.float32),
    )(e, e, e, e)


def kernel(history, current, poi_distance_matrix):
    hist = history.astype(jnp.int32)
    cur = current.astype(jnp.int32)
    e = _sc_energies(hist, cur, poi_distance_matrix)
    return _tc_softmax(e)


# speedup vs baseline: 1.1642x; 1.0081x over previous
import functools

import jax
import jax.numpy as jnp
from jax import lax
from jax.experimental import pallas as pl
from jax.experimental.pallas import tpu as pltpu
from jax.experimental.pallas import tpu_sc as plsc

P = 4096
SEQ = 4096
STATE = 2048
EW = 2048
L = 16
CH = 8
U = 8
BLK = 512


def _sc_body(hist_hbm, cur_hbm, dist_hbm, e_hbm,
             hist_v, cur_all_v, idxa_v, idxb_v, rows_v, ea_v, eb_v,
             sem_in, sem_a, sem_b):
    info = plsc.get_sparse_core_info()
    nc, ns = info.num_cores, info.num_subcores
    nw = nc * ns
    wid = lax.axis_index("s") * nc + lax.axis_index("c")

    pltpu.sync_copy(hist_hbm, hist_v)
    pltpu.sync_copy(cur_hbm, cur_all_v)

    iota = lax.iota(jnp.int32, L)
    NCH = STATE // 32 // CH

    def stage_chunk(c, idx_ref, buf):
        tt = jnp.minimum(c * CH + iota, STATE // nw - 1)
        rowidx = plsc.load_gather(cur_all_v, [wid + tt * nw])
        idx_ref[pl.ds(0, L)] = rowidx
        pltpu.async_copy(dist_hbm.at[idx_ref.at[pl.ds(0, CH)]], buf, sem_in)

    stage_chunk(0, idxa_v, rows_v.at[0])

    def gather_row(t, e_ref):
        c = lax.div(t, CH)
        k = t - c * CH
        i = wid + t * nw
        kvec = jnp.full((L,), k, jnp.int32)
        pvec = jnp.full((L,), c & 1, jnp.int32)

        @pl.when(k == 0)
        def _():
            pltpu.make_async_copy(
                dist_hbm.at[idxa_v.at[pl.ds(0, CH)]], rows_v.at[0],
                sem_in).wait()

            @pl.when((c < NCH - 1) & ((c & 1) == 0))
            def _():
                stage_chunk(c + 1, idxb_v, rows_v.at[1])

            @pl.when((c < NCH - 1) & ((c & 1) == 1))
            def _():
                stage_chunk(c + 1, idxa_v, rows_v.at[0])

        nv2 = (((i + 1) >> 7) << 3) + 16

        @plsc.parallel_loop(0, nv2, unroll=U)
        def _(v):
            idx = hist_v[pl.ds(v * L, L)]
            g = plsc.load_gather(rows_v, [pvec, kvec, idx])
            e_ref[pl.ds(v * L, L)] = 1.0 / g

        return i

    def put_row(i, e_ref, sem):
        for j in range(4):
            @pl.when((i >> 9) == j)
            def _(j=j):
                wj = (j + 1) * 512
                pltpu.async_copy(e_ref.at[pl.ds(0, wj)],
                                 e_hbm.at[i, pl.ds(0, wj)], sem)

    def drain_row(i, e_ref, sem):
        for j in range(4):
            @pl.when((i >> 9) == j)
            def _(j=j):
                wj = (j + 1) * 512
                pltpu.make_async_copy(e_ref.at[pl.ds(0, wj)],
                                      e_hbm.at[i, pl.ds(0, wj)],
                                      sem).wait()

    def pair_body(q, carry):
        ia = gather_row(2 * q, ea_v)
        put_row(ia, ea_v, sem_a)

        @pl.when(q > 0)
        def _():
            drain_row(ia - nw, eb_v, sem_b)

        ib = gather_row(2 * q + 1, eb_v)
        put_row(ib, eb_v, sem_b)
        drain_row(ia, ea_v, sem_a)
        return carry

    lax.fori_loop(0, STATE // nw // 2, pair_body, 0)
    pltpu.make_async_copy(eb_v.at[pl.ds(0, 2048)],
                          e_hbm.at[0, pl.ds(0, 2048)], sem_b).wait()


_sc_energies = functools.partial(
    pl.kernel,
    out_type=jax.ShapeDtypeStruct((STATE, EW), jnp.float32),
    mesh=plsc.VectorSubcoreMesh(core_axis_name="c", subcore_axis_name="s"),
    compiler_params=pltpu.CompilerParams(
        use_tc_tiling_on_sc=True, needs_layout_passes=False),
    scratch_types=[
        pltpu.VMEM((SEQ,), jnp.int32),
        pltpu.VMEM((STATE,), jnp.int32),
        pltpu.VMEM((L,), jnp.int32),
        pltpu.VMEM((L,), jnp.int32),
        pltpu.VMEM((2, CH, SEQ), jnp.float32),
        pltpu.VMEM((SEQ,), jnp.float32),
        pltpu.VMEM((SEQ,), jnp.float32),
        pltpu.SemaphoreType.DMA,
        pltpu.SemaphoreType.DMA,
        pltpu.SemaphoreType.DMA,
    ],
)(_sc_body)


def _tc_softmax_body(e0_ref, e1_ref, e2_ref, e3_ref, o_ref):
    b = pl.program_id(0)
    refs = [e0_ref, e1_ref, e2_ref, e3_ref]
    for j in range(STATE // BLK):
        @pl.when(b == j)
        def _(j=j):
            w = (j + 1) * BLK
            rows = jax.lax.broadcasted_iota(jnp.int32, (BLK, BLK), 0) + j * BLK
            cols = jax.lax.broadcasted_iota(jnp.int32, (BLK, BLK), 1) + j * BLK
            strip = jnp.where(cols <= rows, refs[j][...], 0.0)
            if j > 0:
                e = jnp.concatenate(
                    [refs[t][...] for t in range(j)] + [strip], axis=1)
            else:
                e = strip
            m = jnp.max(e, axis=1, keepdims=True)
            p = jnp.exp(e - m)
            em = jnp.exp(-m)
            s = jnp.sum(p, axis=1, keepdims=True) + float(SEQ - w) * em
            r = 1.0 / s
            o_ref[:, :w] = p * r
            o_ref[:, w:] = jnp.broadcast_to(em * r, (BLK, SEQ - w))


def _make_strip_spec(j):
    return pl.BlockSpec((BLK, BLK), lambda b: (jnp.maximum(b, j), j))


def _tc_softmax(e):
    return pl.pallas_call(
        _tc_softmax_body,
        grid=(STATE // BLK,),
        in_specs=[_make_strip_spec(j) for j in range(STATE // BLK)],
        out_specs=pl.BlockSpec((BLK, SEQ), lambda b: (b, 0)),
        out_shape=jax.ShapeDtypeStruct((STATE, SEQ), jnp.float32),
    )(e, e, e, e)


def kernel(history, current, poi_distance_matrix):
    hist = history.astype(jnp.int32)
    cur = current.astype(jnp.int32)
    e = _sc_energies(hist, cur, poi_distance_matrix)
    return _tc_softmax(e)
